# Initial kernel scaffold; baseline (speedup 1.0000x reference)
import jax
import jax.numpy as jnp
from jax.experimental import pallas as pl


def kernel(x, edge_index, W_l, b_l, W_r):
    raise NotImplementedError("write your pallas kernel here")


# speedup vs baseline: 5.4390x; 5.4390x over previous
import functools

import jax
import jax.numpy as jnp
from jax import lax
from jax.experimental import pallas as pl
from jax.experimental.pallas import tpu as pltpu
from jax.experimental.pallas import tpu_sc as plsc

N_NODES = 10000
N_EDGES = 320000
CH = 128
CHA = 144
NC = 2
NS = 16
E_PER_W = N_EDGES // (NC * NS)
WIN = 80
NWIN = E_PER_W // WIN
ROWS_PER_SUB = 632
N_PAD = NS * ROWS_PER_SUB


def _sc_aggregate(xa, src, dst, zstripe):
    @functools.partial(
        pl.kernel,
        out_type=jax.ShapeDtypeStruct((NC, N_PAD, CHA), jnp.float32),
        mesh=plsc.VectorSubcoreMesh(core_axis_name="c", subcore_axis_name="s"),
        compiler_params=pltpu.CompilerParams(use_tc_tiling_on_sc=False),
        scratch_types=[
            pltpu.VMEM((WIN,), jnp.int32),
            pltpu.VMEM((WIN,), jnp.int32),
            pltpu.VMEM((WIN, CHA), jnp.float32),
            pltpu.VMEM_SHARED((N_PAD, CHA), jnp.float32),
            pltpu.SemaphoreType.DMA,
        ],
    )
    def agg(xa_hbm, src_hbm, dst_hbm, z_hbm, out_hbm, src_v, dst_v, rows_v,
            acc_sh, sem):
        cid = lax.axis_index("c")
        sid = lax.axis_index("s")
        stripe = pl.multiple_of(sid * ROWS_PER_SUB, 8)
        pltpu.sync_copy(z_hbm, acc_sh.at[pl.ds(stripe, ROWS_PER_SUB)])
        plsc.subcore_barrier()

        g0 = (cid * NS + sid) * E_PER_W

        @pl.loop(0, NWIN)
        def _(w):
            base = pl.multiple_of(g0 + w * WIN, 8)
            pltpu.sync_copy(src_hbm.at[pl.ds(base, WIN)], src_v)
            pltpu.sync_copy(dst_hbm.at[pl.ds(base, WIN)], dst_v)
            pltpu.async_copy(xa_hbm.at[src_v], rows_v, sem).wait()
            pltpu.sync_copy(rows_v, acc_sh.at[dst_v], add=True)

        plsc.subcore_barrier()
        pltpu.sync_copy(
            acc_sh.at[pl.ds(stripe, ROWS_PER_SUB)],
            out_hbm.at[cid, pl.ds(stripe, ROWS_PER_SUB)])

    return agg(xa, src, dst, zstripe)


def _tc_finish(acc, x, W_l, b_l, W_r):
    R = 1000

    def body(acc0_ref, acc1_ref, x_ref, wl_ref, bl_ref, wr_ref, o_ref):
        a = acc0_ref[...] + acc1_ref[...]
        summed = a[:, :CH]
        counts = a[:, CH:CH + 1]
        mean = summed / jnp.maximum(counts, 1.0)
        z = jnp.dot(mean, wl_ref[...], preferred_element_type=jnp.float32)
        z = z + bl_ref[...] + jnp.dot(x_ref[...], wr_ref[...],
                                      preferred_element_type=jnp.float32)
        o_ref[...] = jnp.maximum(z, 0.0)

    return pl.pallas_call(
        body,
        grid=(N_NODES // R,),
        in_specs=[
            pl.BlockSpec((R, CHA), lambda i: (i, 0)),
            pl.BlockSpec((R, CHA), lambda i: (i, 0)),
            pl.BlockSpec((R, CH), lambda i: (i, 0)),
            pl.BlockSpec((CH, CH), lambda i: (0, 0)),
            pl.BlockSpec((1, CH), lambda i: (0, 0)),
            pl.BlockSpec((CH, CH), lambda i: (0, 0)),
        ],
        out_specs=pl.BlockSpec((R, CH), lambda i: (i, 0)),
        out_shape=jax.ShapeDtypeStruct((N_NODES, CH), jnp.float32),
    )(acc[0], acc[1], x, W_l, b_l.reshape(1, CH), W_r)


def kernel(x, edge_index, W_l, b_l, W_r):
    src = edge_index[0]
    dst = edge_index[1]
    xa = jnp.concatenate(
        [x, jnp.ones((N_NODES, 1), jnp.float32),
         jnp.zeros((N_NODES, CHA - CH - 1), jnp.float32)], axis=1)
    zstripe = jnp.zeros((ROWS_PER_SUB, CHA), jnp.float32)
    acc = _sc_aggregate(xa, src, dst, zstripe)[:, :N_NODES]
    return _tc_finish(acc, x, W_l, b_l, W_r)


# speedup vs baseline: 8.3556x; 1.5362x over previous
import functools

import jax
import jax.numpy as jnp
from jax import lax
from jax.experimental import pallas as pl
from jax.experimental.pallas import tpu as pltpu
from jax.experimental.pallas import tpu_sc as plsc

N_NODES = 10000
N_EDGES = 320000
CH = 128
CHA = 144
NC = 2
NS = 16
E_PER_W = N_EDGES // (NC * NS)
WIN = 80
NWIN = E_PER_W // WIN
ROWS_PER_SUB = 632
N_PAD = NS * ROWS_PER_SUB


def _sc_aggregate(xa, src, dst, zstripe):
    @functools.partial(
        pl.kernel,
        out_type=jax.ShapeDtypeStruct((NC, N_PAD, CHA), jnp.float32),
        mesh=plsc.VectorSubcoreMesh(core_axis_name="c", subcore_axis_name="s"),
        compiler_params=pltpu.CompilerParams(use_tc_tiling_on_sc=False),
        scratch_types=[
            pltpu.VMEM((WIN,), jnp.int32),
            pltpu.VMEM((WIN,), jnp.int32),
            pltpu.VMEM((WIN,), jnp.int32),
            pltpu.VMEM((WIN,), jnp.int32),
            pltpu.VMEM((WIN, CHA), jnp.float32),
            pltpu.VMEM((WIN, CHA), jnp.float32),
            pltpu.VMEM_SHARED((N_PAD, CHA), jnp.float32),
            pltpu.SemaphoreType.DMA,
            pltpu.SemaphoreType.DMA,
        ],
    )
    def agg(xa_hbm, src_hbm, dst_hbm, z_hbm, out_hbm, src_v0, src_v1,
            dst_v0, dst_v1, rows_v0, rows_v1, acc_sh, sem0, sem1):
        cid = lax.axis_index("c")
        sid = lax.axis_index("s")
        stripe = pl.multiple_of(sid * ROWS_PER_SUB, 8)
        pltpu.sync_copy(z_hbm, acc_sh.at[pl.ds(stripe, ROWS_PER_SUB)])
        plsc.subcore_barrier()

        g0 = (cid * NS + sid) * E_PER_W
        srcs = (src_v0, src_v1)
        dsts = (dst_v0, dst_v1)
        rows = (rows_v0, rows_v1)
        sems = (sem0, sem1)

        def fire(w, b):
            base = pl.multiple_of(g0 + w * WIN, 8)
            pltpu.sync_copy(src_hbm.at[pl.ds(base, WIN)], srcs[b])
            pltpu.sync_copy(dst_hbm.at[pl.ds(base, WIN)], dsts[b])
            pltpu.async_copy(xa_hbm.at[srcs[b]], rows[b], sems[b])

        def drain(b):
            pltpu.make_async_copy(xa_hbm.at[srcs[b]], rows[b], sems[b]).wait()
            pltpu.sync_copy(rows[b], acc_sh.at[dsts[b]], add=True)

        fire(0, 0)

        @pl.loop(0, NWIN - 1, step=2)
        def _(w):
            fire(w + 1, 1)
            drain(0)
            fire(w + 2, 0)
            drain(1)

        drain(0)
        plsc.subcore_barrier()
        pltpu.sync_copy(
            acc_sh.at[pl.ds(stripe, ROWS_PER_SUB)],
            out_hbm.at[cid, pl.ds(stripe, ROWS_PER_SUB)])

    return agg(xa, src, dst, zstripe)


def _tc_finish(acc, x, W_l, b_l, W_r):
    R = 1000

    def body(acc_ref, x_ref, wl_ref, bl_ref, wr_ref, o_ref):
        a = acc_ref[0] + acc_ref[1]
        summed = a[:, :CH]
        counts = a[:, CH:CH + 1]
        mean = summed / jnp.maximum(counts, 1.0)
        z = jnp.dot(mean, wl_ref[...], preferred_element_type=jnp.float32)
        z = z + bl_ref[...] + jnp.dot(x_ref[...], wr_ref[...],
                                      preferred_element_type=jnp.float32)
        o_ref[...] = jnp.maximum(z, 0.0)

    return pl.pallas_call(
        body,
        grid=(N_NODES // R,),
        in_specs=[
            pl.BlockSpec((NC, R, CHA), lambda i: (0, i, 0)),
            pl.BlockSpec((R, CH), lambda i: (i, 0)),
            pl.BlockSpec((CH, CH), lambda i: (0, 0)),
            pl.BlockSpec((1, CH), lambda i: (0, 0)),
            pl.BlockSpec((CH, CH), lambda i: (0, 0)),
        ],
        out_specs=pl.BlockSpec((R, CH), lambda i: (i, 0)),
        out_shape=jax.ShapeDtypeStruct((N_NODES, CH), jnp.float32),
    )(acc, x, W_l, b_l.reshape(1, CH), W_r)


def kernel(x, edge_index, W_l, b_l, W_r):
    src = edge_index[0]
    dst = edge_index[1]
    xa = jnp.concatenate(
        [x, jnp.ones((N_NODES, 1), jnp.float32),
         jnp.zeros((N_NODES, CHA - CH - 1), jnp.float32)], axis=1)
    zstripe = jnp.zeros((ROWS_PER_SUB, CHA), jnp.float32)
    acc = _sc_aggregate(xa, src, dst, zstripe)
    return _tc_finish(acc, x, W_l, b_l, W_r)


# speedup vs baseline: 9.1152x; 1.0909x over previous
import functools

import jax
import jax.numpy as jnp
from jax import lax
from jax.experimental import pallas as pl
from jax.experimental.pallas import tpu as pltpu
from jax.experimental.pallas import tpu_sc as plsc

N_NODES = 10000
N_EDGES = 320000
CH = 128
CNTW = 16
NC = 2
NS = 16
E_PER_W = N_EDGES // (NC * NS)
WIN = 80
NWIN = E_PER_W // WIN
ROWS_PER_SUB = 632
N_PAD = NS * ROWS_PER_SUB


def _sc_aggregate(x, src, dst, zf, zc):
    @functools.partial(
        pl.kernel,
        out_type=(
            jax.ShapeDtypeStruct((NC, N_PAD, CH), jnp.float32),
            jax.ShapeDtypeStruct((NC, N_PAD, CNTW), jnp.float32),
        ),
        mesh=plsc.VectorSubcoreMesh(core_axis_name="c", subcore_axis_name="s"),
        compiler_params=pltpu.CompilerParams(use_tc_tiling_on_sc=False),
        scratch_types=[
            pltpu.VMEM((WIN,), jnp.int32),
            pltpu.VMEM((WIN,), jnp.int32),
            pltpu.VMEM((WIN,), jnp.int32),
            pltpu.VMEM((WIN,), jnp.int32),
            pltpu.VMEM((WIN, CH), jnp.float32),
            pltpu.VMEM((WIN, CH), jnp.float32),
            pltpu.VMEM((WIN, CNTW), jnp.float32),
            pltpu.VMEM_SHARED((N_PAD, CH), jnp.float32),
            pltpu.VMEM_SHARED((N_PAD, CNTW), jnp.float32),
            pltpu.SemaphoreType.DMA,
            pltpu.SemaphoreType.DMA,
        ],
    )
    def agg(x_hbm, src_hbm, dst_hbm, zf_hbm, zc_hbm, out_hbm, cnt_hbm,
            src_v0, src_v1, dst_v0, dst_v1, rows_v0, rows_v1, ones_v,
            acc_sh, cnt_sh, sem0, sem1):
        cid = lax.axis_index("c")
        sid = lax.axis_index("s")
        stripe = pl.multiple_of(sid * ROWS_PER_SUB, 8)
        pltpu.sync_copy(zf_hbm, acc_sh.at[pl.ds(stripe, ROWS_PER_SUB)])
        pltpu.sync_copy(zc_hbm, cnt_sh.at[pl.ds(stripe, ROWS_PER_SUB)])

        @pl.loop(0, WIN)
        def _(i):
            ones_v[i, :] = jnp.ones((CNTW,), jnp.float32)

        plsc.subcore_barrier()

        g0 = (cid * NS + sid) * E_PER_W
        srcs = (src_v0, src_v1)
        dsts = (dst_v0, dst_v1)
        rows = (rows_v0, rows_v1)
        sems = (sem0, sem1)

        def fire(w, b):
            base = pl.multiple_of(g0 + w * WIN, 8)
            pltpu.sync_copy(src_hbm.at[pl.ds(base, WIN)], srcs[b])
            pltpu.sync_copy(dst_hbm.at[pl.ds(base, WIN)], dsts[b])
            pltpu.async_copy(x_hbm.at[srcs[b]], rows[b], sems[b])

        def drain(b):
            pltpu.make_async_copy(x_hbm.at[srcs[b]], rows[b], sems[b]).wait()
            pltpu.sync_copy(rows[b], acc_sh.at[dsts[b]], add=True)
            pltpu.sync_copy(ones_v, cnt_sh.at[dsts[b]], add=True)

        fire(0, 0)

        @pl.loop(0, NWIN - 1, step=2)
        def _(w):
            fire(w + 1, 1)
            drain(0)
            fire(w + 2, 0)
            drain(1)

        drain(0)
        plsc.subcore_barrier()
        pltpu.sync_copy(
            acc_sh.at[pl.ds(stripe, ROWS_PER_SUB)],
            out_hbm.at[cid, pl.ds(stripe, ROWS_PER_SUB)])
        pltpu.sync_copy(
            cnt_sh.at[pl.ds(stripe, ROWS_PER_SUB)],
            cnt_hbm.at[cid, pl.ds(stripe, ROWS_PER_SUB)])

    return agg(x, src, dst, zf, zc)


def _tc_finish(acc, cnt, x, W_l, b_l, W_r):
    R = 1000

    def body(acc_ref, cnt_ref, x_ref, wl_ref, bl_ref, wr_ref, o_ref):
        summed = acc_ref[0] + acc_ref[1]
        counts = (cnt_ref[0] + cnt_ref[1])[:, :1]
        mean = summed / jnp.maximum(counts, 1.0)
        z = jnp.dot(mean, wl_ref[...], preferred_element_type=jnp.float32)
        z = z + bl_ref[...] + jnp.dot(x_ref[...], wr_ref[...],
                                      preferred_element_type=jnp.float32)
        o_ref[...] = jnp.maximum(z, 0.0)

    return pl.pallas_call(
        body,
        grid=(N_NODES // R,),
        in_specs=[
            pl.BlockSpec((NC, R, CH), lambda i: (0, i, 0)),
            pl.BlockSpec((NC, R, CNTW), lambda i: (0, i, 0)),
            pl.BlockSpec((R, CH), lambda i: (i, 0)),
            pl.BlockSpec((CH, CH), lambda i: (0, 0)),
            pl.BlockSpec((1, CH), lambda i: (0, 0)),
            pl.BlockSpec((CH, CH), lambda i: (0, 0)),
        ],
        out_specs=pl.BlockSpec((R, CH), lambda i: (i, 0)),
        out_shape=jax.ShapeDtypeStruct((N_NODES, CH), jnp.float32),
    )(acc, cnt, x, W_l, b_l.reshape(1, CH), W_r)


def kernel(x, edge_index, W_l, b_l, W_r):
    src = edge_index[0]
    dst = edge_index[1]
    zf = jnp.zeros((ROWS_PER_SUB, CH), jnp.float32)
    zc = jnp.zeros((ROWS_PER_SUB, CNTW), jnp.float32)
    acc, cnt = _sc_aggregate(x, src, dst, zf, zc)
    return _tc_finish(acc, cnt, x, W_l, b_l, W_r)


# speedup vs baseline: 13.5883x; 1.4907x over previous
import functools

import jax
import jax.numpy as jnp
from jax import lax
from jax.experimental import pallas as pl
from jax.experimental.pallas import tpu as pltpu
from jax.experimental.pallas import tpu_sc as plsc

N_NODES = 10000
N_EDGES = 320000
CH = 128
CNTW = 16
NC = 2
NS = 16
NW = NC * NS
E_PER_W = N_EDGES // NW
WIN = 80
NWIN = E_PER_W // WIN
CWIN = 25
NCHUNK = NWIN // CWIN
ROWS_PER_SUB = 632
N_PAD = NS * ROWS_PER_SUB


def _sc_aggregate(x, src3, dst3, zf, zc):
    @functools.partial(
        pl.kernel,
        out_type=(
            jax.ShapeDtypeStruct((NC, N_PAD, CH), jnp.float32),
            jax.ShapeDtypeStruct((NC, N_PAD, CNTW), jnp.float32),
        ),
        mesh=plsc.VectorSubcoreMesh(core_axis_name="c", subcore_axis_name="s"),
        compiler_params=pltpu.CompilerParams(use_tc_tiling_on_sc=False),
        scratch_types=[
            pltpu.VMEM((CWIN, WIN), jnp.int32),
            pltpu.VMEM((CWIN, WIN), jnp.int32),
            pltpu.VMEM((WIN, CH), jnp.float32),
            pltpu.VMEM((WIN, CH), jnp.float32),
            pltpu.VMEM((WIN, CH), jnp.float32),
            pltpu.VMEM((WIN, CNTW), jnp.float32),
            pltpu.VMEM_SHARED((N_PAD, CH), jnp.float32),
            pltpu.VMEM_SHARED((N_PAD, CNTW), jnp.float32),
            pltpu.SemaphoreType.DMA,
            pltpu.SemaphoreType.DMA,
            pltpu.SemaphoreType.DMA,
            pltpu.SemaphoreType.DMA,
            pltpu.SemaphoreType.DMA,
            pltpu.SemaphoreType.DMA,
            pltpu.SemaphoreType.DMA,
            pltpu.SemaphoreType.DMA,
            pltpu.SemaphoreType.DMA,
        ],
    )
    def agg(x_hbm, src_hbm, dst_hbm, zf_hbm, zc_hbm, out_hbm, cnt_hbm,
            src_v, dst_v, rows_v0, rows_v1, rows_v2, ones_v, acc_sh, cnt_sh,
            g0, g1, g2, s0, s1, s2, c0, c1, c2):
        cid = lax.axis_index("c")
        sid = lax.axis_index("s")
        wid = cid * NS + sid
        stripe = pl.multiple_of(sid * ROWS_PER_SUB, 8)
        rows = (rows_v0, rows_v1, rows_v2)
        gsem = (g0, g1, g2)
        ssem = (s0, s1, s2)
        csem = (c0, c1, c2)

        zf_cp = pltpu.async_copy(
            zf_hbm, acc_sh.at[pl.ds(stripe, ROWS_PER_SUB)], g0)
        zc_cp = pltpu.async_copy(
            zc_hbm, cnt_sh.at[pl.ds(stripe, ROWS_PER_SUB)], g1)
        pltpu.sync_copy(src_hbm.at[wid], src_v)
        pltpu.sync_copy(dst_hbm.at[wid], dst_v)

        @pl.loop(0, WIN)
        def _(i):
            ones_v[i, :] = jnp.ones((CNTW,), jnp.float32)

        zf_cp.wait()
        zc_cp.wait()
        plsc.subcore_barrier()

        def fire_gather(w, b):
            pltpu.async_copy(x_hbm.at[src_v.at[w]], rows[b], gsem[b])

        def wait_gather(w, b):
            pltpu.make_async_copy(
                x_hbm.at[src_v.at[w]], rows[b], gsem[b]).wait()

        def start_scatter(w, b):
            pltpu.async_copy(rows[b], acc_sh.at[dst_v.at[w]], ssem[b],
                             add=True)
            pltpu.async_copy(ones_v, cnt_sh.at[dst_v.at[w]], csem[b],
                             add=True)

        def wait_scatter(w, b):
            pltpu.make_async_copy(rows[b], acc_sh.at[dst_v.at[w]],
                                  ssem[b]).wait()
            pltpu.make_async_copy(ones_v, cnt_sh.at[dst_v.at[w]],
                                  csem[b]).wait()

        for k in range(NCHUNK):
            pltpu.sync_copy(src_hbm.at[wid * NCHUNK + k], src_v)
            pltpu.sync_copy(dst_hbm.at[wid * NCHUNK + k], dst_v)

            fire_gather(0, 0)
            fire_gather(1, 1)
            wait_gather(0, 0)
            start_scatter(0, 0)
            fire_gather(2, 2)
            wait_gather(1, 1)
            start_scatter(1, 1)
            wait_scatter(0, 0)
            fire_gather(3, 0)

            @pl.loop(0, (CWIN - 4) // 3)
            def _(q):
                for db in range(3):
                    w = 2 + q * 3 + db
                    b = (2 + db) % 3
                    wait_gather(w, b)
                    start_scatter(w, b)
                    wait_scatter(w - 1, (b + 2) % 3)
                    fire_gather(w + 2, (b + 2) % 3)

            wait_gather(CWIN - 2, (CWIN - 2) % 3)
            start_scatter(CWIN - 2, (CWIN - 2) % 3)
            wait_scatter(CWIN - 3, (CWIN - 3) % 3)
            wait_gather(CWIN - 1, (CWIN - 1) % 3)
            start_scatter(CWIN - 1, (CWIN - 1) % 3)
            wait_scatter(CWIN - 2, (CWIN - 2) % 3)
            wait_scatter(CWIN - 1, (CWIN - 1) % 3)

        plsc.subcore_barrier()
        pltpu.sync_copy(
            acc_sh.at[pl.ds(stripe, ROWS_PER_SUB)],
            out_hbm.at[cid, pl.ds(stripe, ROWS_PER_SUB)])
        pltpu.sync_copy(
            cnt_sh.at[pl.ds(stripe, ROWS_PER_SUB)],
            cnt_hbm.at[cid, pl.ds(stripe, ROWS_PER_SUB)])

    return agg(x, src3, dst3, zf, zc)


def _tc_finish(acc, cnt, x, W_l, b_l, W_r):
    R = 1000

    def body(acc_ref, cnt_ref, x_ref, wl_ref, bl_ref, wr_ref, o_ref):
        summed = acc_ref[0] + acc_ref[1]
        counts = (cnt_ref[0] + cnt_ref[1])[:, :1]
        mean = summed / jnp.maximum(counts, 1.0)
        z = jnp.dot(mean, wl_ref[...], preferred_element_type=jnp.float32)
        z = z + bl_ref[...] + jnp.dot(x_ref[...], wr_ref[...],
                                      preferred_element_type=jnp.float32)
        o_ref[...] = jnp.maximum(z, 0.0)

    return pl.pallas_call(
        body,
        grid=(N_NODES // R,),
        in_specs=[
            pl.BlockSpec((NC, R, CH), lambda i: (0, i, 0)),
            pl.BlockSpec((NC, R, CNTW), lambda i: (0, i, 0)),
            pl.BlockSpec((R, CH), lambda i: (i, 0)),
            pl.BlockSpec((CH, CH), lambda i: (0, 0)),
            pl.BlockSpec((1, CH), lambda i: (0, 0)),
            pl.BlockSpec((CH, CH), lambda i: (0, 0)),
        ],
        out_specs=pl.BlockSpec((R, CH), lambda i: (i, 0)),
        out_shape=jax.ShapeDtypeStruct((N_NODES, CH), jnp.float32),
    )(acc, cnt, x, W_l, b_l.reshape(1, CH), W_r)


def kernel(x, edge_index, W_l, b_l, W_r):
    src3 = edge_index[0].reshape(NW * NCHUNK, CWIN, WIN)
    dst3 = edge_index[1].reshape(NW * NCHUNK, CWIN, WIN)
    zf = jnp.zeros((ROWS_PER_SUB, CH), jnp.float32)
    zc = jnp.zeros((ROWS_PER_SUB, CNTW), jnp.float32)
    acc, cnt = _sc_aggregate(x, src3, dst3, zf, zc)
    return _tc_finish(acc, cnt, x, W_l, b_l, W_r)
